# baseline (speedup 1.0000x reference)
import functools

import jax
import jax.numpy as jnp
from jax import lax
from jax.experimental import pallas as pl
from jax.experimental.pallas import tpu as pltpu
from jax.experimental.pallas import tpu_sc as plsc

_K = 1024
_D = 64
_ROWS = 18432
_R = 1024
_NBLK = _ROWS // _R

_NW = 32
_BPW = _ROWS // _NW
_CH = 96
_NCH = _BPW // _CH


def _argmin_body(x_ref, cb2_ref, cn_ref, rsq_ref, idx_ref, loss_ref):
    xb = x_ref[...]
    cb2 = cb2_ref[...]
    s2 = lax.dot_general(xb, cb2, (((1,), (1,)), ((), ())),
                         preferred_element_type=jnp.float32)
    d = (rsq_ref[...] - s2) + cn_ref[...]
    m = jnp.min(d, axis=1, keepdims=True)
    ii = lax.broadcasted_iota(jnp.int32, d.shape, 1)
    idx = jnp.min(jnp.where(d == m, ii, _K), axis=1)
    idx_ref[...] = idx.reshape(_R // 128, 128)

    @pl.when(pl.program_id(0) == 0)
    def _():
        loss_ref[...] = jnp.zeros((1, 1), jnp.float32)

    loss_ref[...] += jnp.sum(m, axis=(0, 1), keepdims=True)


_argmin_call = pl.pallas_call(
    _argmin_body,
    grid=(_NBLK,),
    in_specs=[
        pl.BlockSpec((_R, _D), lambda i: (i, 0)),
        pl.BlockSpec((_K, _D), lambda i: (0, 0)),
        pl.BlockSpec((1, _K), lambda i: (0, 0)),
        pl.BlockSpec((_R, 1), lambda i: (i, 0)),
    ],
    out_specs=[
        pl.BlockSpec((_R // 128, 128), lambda i: (i, 0)),
        pl.BlockSpec((1, 1), lambda i: (0, 0)),
    ],
    out_shape=[
        jax.ShapeDtypeStruct((_ROWS // 128, 128), jnp.int32),
        jax.ShapeDtypeStruct((1, 1), jnp.float32),
    ],
)


@functools.partial(
    pl.kernel,
    mesh=plsc.VectorSubcoreMesh(core_axis_name="c", subcore_axis_name="s"),
    out_type=jax.ShapeDtypeStruct((_ROWS, 2 * _D), jnp.float32),
    scratch_types=[
        pltpu.VMEM((_BPW,), jnp.int32),
        pltpu.VMEM((_BPW, 2 * _D), jnp.float32),
        pltpu.SemaphoreType.DMA,
    ],
)
def _sc_gather(cbp_hbm, idx_hbm, out_hbm, idx_v, rows_v, sem):
    wid = lax.axis_index("c") * 16 + lax.axis_index("s")
    base = wid * _BPW
    pltpu.sync_copy(idx_hbm.at[pl.ds(base, _BPW)], idx_v)
    copies = [
        pltpu.async_copy(cbp_hbm.at[idx_v.at[pl.ds(j * _CH, _CH)]],
                         rows_v.at[pl.ds(j * _CH, _CH)], sem)
        for j in range(_NCH)
    ]
    for c in copies:
        c.wait()
    pltpu.sync_copy(rows_v, out_hbm.at[pl.ds(base, _BPW)])


def kernel(x, codebook):
    B, T, D = x.shape
    flat = x.reshape(-1, D)
    cn = jnp.sum(codebook ** 2, axis=1)[None, :]
    rsq = jnp.sum(flat ** 2, axis=1, keepdims=True)
    cbp = jnp.pad(codebook, ((0, 0), (0, _D)))
    idx2d, loss_sum = _argmin_call(flat, 2.0 * codebook, cn, rsq)
    idx_flat = idx2d.reshape(-1)
    zq_pad = _sc_gather(cbp, idx_flat)
    zq_st = zq_pad.reshape(B, T, 2 * D)[:, :, :D]
    loss = 1.25 * loss_sum[0, 0] / (B * T * D)
    return zq_st, loss, idx_flat.reshape(B, T)


# speedup vs baseline: 1.1177x; 1.1177x over previous
import functools

import jax
import jax.numpy as jnp
from jax import lax
from jax.experimental import pallas as pl
from jax.experimental.pallas import tpu as pltpu
from jax.experimental.pallas import tpu_sc as plsc

_K = 1024
_D = 64
_ROWS = 18432
_R = 1024
_NBLK = _ROWS // _R

_NW = 32
_T = _ROWS // _NW
_LANES = 16


def _argmin_body(x_ref, cb_ref, cn_ref, rsq_ref, idx_ref, loss_ref):
    xb = x_ref[...]
    cb = cb_ref[...]
    s2 = lax.dot_general(xb + xb, cb, (((1,), (1,)), ((), ())),
                         preferred_element_type=jnp.float32)
    d = (rsq_ref[...] - s2) + cn_ref[...]
    m = jnp.min(d, axis=1, keepdims=True)
    ii = lax.broadcasted_iota(jnp.int32, d.shape, 1)
    idx = jnp.min(jnp.where(d == m, ii, _K), axis=1)
    idx_ref[...] = idx.reshape(_R // 128, 128)

    @pl.when(pl.program_id(0) == 0)
    def _():
        loss_ref[...] = jnp.zeros((1, 1), jnp.float32)

    loss_ref[...] += jnp.sum(m, axis=(0, 1), keepdims=True)

    @pl.when(pl.program_id(0) == _NBLK - 1)
    def _():
        loss_ref[...] = loss_ref[...] * (1.25 / (_ROWS * _D))


_argmin_call = pl.pallas_call(
    _argmin_body,
    grid=(_NBLK,),
    in_specs=[
        pl.BlockSpec((_R, _D), lambda i: (i, 0)),
        pl.BlockSpec((_K, _D), lambda i: (0, 0)),
        pl.BlockSpec((1, _K), lambda i: (0, 0)),
        pl.BlockSpec((_R, 1), lambda i: (i, 0)),
    ],
    out_specs=[
        pl.BlockSpec((_R // 128, 128), lambda i: (i, 0)),
        pl.BlockSpec((1, 1), lambda i: (0, 0)),
    ],
    out_shape=[
        jax.ShapeDtypeStruct((_ROWS // 128, 128), jnp.int32),
        jax.ShapeDtypeStruct((1, 1), jnp.float32),
    ],
)


@functools.partial(
    pl.kernel,
    mesh=plsc.VectorSubcoreMesh(core_axis_name="c", subcore_axis_name="s"),
    compiler_params=pltpu.CompilerParams(needs_layout_passes=False),
    out_type=jax.ShapeDtypeStruct((_NW, _D, _T), jnp.float32),
    scratch_types=[
        pltpu.VMEM((_D, _K), jnp.float32),
        pltpu.VMEM((_T,), jnp.int32),
        pltpu.VMEM((_D, _T), jnp.float32),
    ],
)
def _sc_gather_t(cbt_hbm, idx_hbm, out_hbm, cbt_v, idx_v, zqt_v):
    w = lax.axis_index("c") * 16 + lax.axis_index("s")
    pltpu.sync_copy(cbt_hbm, cbt_v)
    pltpu.sync_copy(idx_hbm.at[pl.ds(w * _T, _T)], idx_v)
    nch = _T // _LANES
    idx_chunks = [idx_v[pl.ds(tc * _LANES, _LANES)] for tc in range(nch)]

    def body(dd, carry):
        row = jnp.full((_LANES,), dd, jnp.int32)
        for tc in range(nch):
            vals = plsc.load_gather(cbt_v, [row, idx_chunks[tc]])
            zqt_v[dd, pl.ds(tc * _LANES, _LANES)] = vals
        return carry

    lax.fori_loop(0, _D, body, 0)
    pltpu.sync_copy(zqt_v, out_hbm.at[w])


def kernel(x, codebook):
    B, T, D = x.shape
    flat = x.reshape(-1, D)
    cn = jnp.sum(codebook ** 2, axis=1)[None, :]
    rsq = jnp.sum(flat ** 2, axis=1, keepdims=True)
    idx2d, loss = _argmin_call(flat, codebook, cn, rsq)
    idx_flat = idx2d.reshape(-1)
    zqt = _sc_gather_t(jnp.swapaxes(codebook, 0, 1), idx_flat)
    zq_st = jnp.swapaxes(zqt, 1, 2)
    return zq_st, loss.reshape(()), idx_flat.reshape(B, T)


# speedup vs baseline: 1.4909x; 1.3339x over previous
import functools

import jax
import jax.numpy as jnp
from jax import lax
from jax.experimental import pallas as pl
from jax.experimental.pallas import tpu as pltpu
from jax.experimental.pallas import tpu_sc as plsc

_K = 1024
_D = 64
_B = 32
_T = 576
_ROWS = _B * _T
_BPS = 16
_NBLK = _B // _BPS

_NW = 32
_LANES = 16


def _argmin_body(xt_ref, cbt_ref, cn_ref, rsq_ref, idx_ref, loss_ref):
    cbt = cbt_ref[...]
    cn = cn_ref[...]
    msum = jnp.zeros((1, 1), jnp.float32)
    parts = []
    for b in range(_BPS):
        xtb = xt_ref[b]
        s2 = lax.dot_general(cbt, xtb + xtb, (((0,), (0,)), ((), ())),
                             preferred_element_type=jnp.float32)
        d = (rsq_ref[b] - s2) + cn
        m = jnp.min(d, axis=0, keepdims=True)
        ii = lax.broadcasted_iota(jnp.int32, d.shape, 0)
        parts.append(jnp.min(jnp.where(d == m, ii, _K), axis=0))
        msum = msum + jnp.sum(m, axis=(0, 1), keepdims=True)
    idx_ref[...] = jnp.concatenate(parts).reshape(_BPS * _T // 128, 128)

    @pl.when(pl.program_id(0) == 0)
    def _():
        loss_ref[...] = jnp.zeros((1, 1), jnp.float32)

    loss_ref[...] += msum

    @pl.when(pl.program_id(0) == _NBLK - 1)
    def _():
        loss_ref[...] = loss_ref[...] * (1.25 / (_ROWS * _D))


_argmin_call = pl.pallas_call(
    _argmin_body,
    grid=(_NBLK,),
    in_specs=[
        pl.BlockSpec((_BPS, _D, _T), lambda i: (i, 0, 0)),
        pl.BlockSpec((_D, _K), lambda i: (0, 0)),
        pl.BlockSpec((_K, 1), lambda i: (0, 0)),
        pl.BlockSpec((_BPS, 1, _T), lambda i: (i, 0, 0)),
    ],
    out_specs=[
        pl.BlockSpec((_BPS * _T // 128, 128), lambda i: (i, 0)),
        pl.BlockSpec((1, 1), lambda i: (0, 0)),
    ],
    out_shape=[
        jax.ShapeDtypeStruct((_ROWS // 128, 128), jnp.int32),
        jax.ShapeDtypeStruct((1, 1), jnp.float32),
    ],
)


@functools.partial(
    pl.kernel,
    mesh=plsc.VectorSubcoreMesh(core_axis_name="c", subcore_axis_name="s"),
    compiler_params=pltpu.CompilerParams(needs_layout_passes=False),
    out_type=jax.ShapeDtypeStruct((_NW, _D, _T), jnp.float32),
    scratch_types=[
        pltpu.VMEM((_D, _K), jnp.float32),
        pltpu.VMEM((_T,), jnp.int32),
        pltpu.VMEM((_D, _T), jnp.float32),
    ],
)
def _sc_gather_t(cbt_hbm, idx_hbm, out_hbm, cbt_v, idx_v, zqt_v):
    w = lax.axis_index("c") * 16 + lax.axis_index("s")
    pltpu.sync_copy(cbt_hbm, cbt_v)
    pltpu.sync_copy(idx_hbm.at[pl.ds(w * _T, _T)], idx_v)
    nch = _T // _LANES
    idx_chunks = [idx_v[pl.ds(tc * _LANES, _LANES)] for tc in range(nch)]

    def body(dd, carry):
        row = jnp.full((_LANES,), dd, jnp.int32)
        for tc in range(nch):
            vals = plsc.load_gather(cbt_v, [row, idx_chunks[tc]])
            zqt_v[dd, pl.ds(tc * _LANES, _LANES)] = vals
        return carry

    lax.fori_loop(0, _D, body, 0)
    pltpu.sync_copy(zqt_v, out_hbm.at[w])


def kernel(x, codebook):
    B, T, D = x.shape
    cbt = jnp.swapaxes(codebook, 0, 1)
    xt = jnp.swapaxes(x, 1, 2)
    cn = jnp.sum(codebook ** 2, axis=1)[:, None]
    rsq = jnp.sum(x ** 2, axis=2)[:, None, :]
    idx2d, loss = _argmin_call(xt, cbt, cn, rsq)
    idx_flat = idx2d.reshape(-1)
    zqt = _sc_gather_t(cbt, idx_flat)
    zq_st = jnp.swapaxes(zqt, 1, 2)
    return zq_st, loss.reshape(()), idx_flat.reshape(B, T)
